# baseline (speedup 1.0000x reference)
import jax
import jax.numpy as jnp
from jax import lax
from jax.experimental import pallas as pl
from jax.experimental.pallas import tpu as pltpu
from jax.experimental.pallas import tpu_sc as plsc

PAD_ID = 0
LANES = 16
NUM_CORES = 2
NUM_SUBCORES = 16
NUM_WORKERS = NUM_CORES * NUM_SUBCORES


def _sc_scorer(B, L, V, DP, LP2):
    BPW = B // NUM_WORKERS
    NCHUNK = DP // LANES
    INV_PAD = ((L + 1 + 63) // 64) * 64

    mesh = plsc.VectorSubcoreMesh(
        core_axis_name="c", subcore_axis_name="s")

    def body(tok_hbm, table_hbm, wmat_hbm, inv_hbm, out_hbm,
             idx_v, rowbuf0, rowbuf1, wv, inv_v, stage, sem0, sem1):
        wid = lax.axis_index("s") * NUM_CORES + lax.axis_index("c")
        base = wid * BPW
        pltpu.sync_copy(wmat_hbm, wv)
        pltpu.sync_copy(inv_hbm, inv_v)
        pltpu.sync_copy(tok_hbm.at[pl.ds(base * LP2, BPW * LP2)], idx_v)

        lane = jnp.arange(LANES, dtype=jnp.int32)
        cols = [lane + j * LANES for j in range(NCHUNK)]
        sems = (sem0, sem1)
        bufs = (rowbuf0, rowbuf1)
        wchunks = [[plsc.load_gather(wv, [lane + (a * NCHUNK + j) * LANES])
                    for j in range(NCHUNK)] for a in (0, 1)]

        def gather(b, slot):
            pltpu.async_copy(table_hbm.at[idx_v.at[pl.ds(b * LP2, LP2)]],
                             bufs[slot], sems[slot])

        def wait(b, slot):
            pltpu.make_async_copy(table_hbm.at[idx_v.at[pl.ds(b * LP2, LP2)]],
                                  bufs[slot], sems[slot]).wait()

        def half(a, b, slot):
            rb = bufs[slot]
            off = b * LP2 + a * L

            def tok_body(l, carry):
                tokv = plsc.load_gather(idx_v, [jnp.zeros(
                    (LANES,), jnp.int32) + (off + l)])
                nonpad = tokv != PAD_ID
                mf = jnp.where(nonpad, 1.0, 0.0)
                lvec = jnp.zeros((LANES,), jnp.int32) + (a * L + l)
                new = [carry[0] + jnp.where(nonpad, 1, 0).astype(jnp.int32)]
                for j in range(NCHUNK):
                    v = plsc.load_gather(rb, [lvec, cols[j]])
                    new.append(carry[1 + j] + mf * v)
                return tuple(new)

            init = ((jnp.zeros((LANES,), jnp.int32),) + tuple(
                jnp.zeros((LANES,), jnp.float32) for _ in range(NCHUNK)))
            res = lax.fori_loop(0, L, tok_body, init)
            inv = plsc.load_gather(inv_v, [res[0]])
            s = res[1] * wchunks[a][0]
            for j in range(1, NCHUNK):
                s = s + res[1 + j] * wchunks[a][j]
            return (jnp.zeros((LANES,), jnp.float32) + jnp.sum(s)) * inv

        def do_row(b, slot):
            wait(b, slot)
            score = half(0, b, slot) + half(1, b, slot)

            @pl.when(b + 2 < BPW)
            def _():
                gather(b + 2, slot)

            plsc.store_scatter(stage, [b * LANES + lane], score)

        gather(0, 0)
        gather(1, 1)

        def g_body(g, carry):
            do_row(2 * g, 0)
            do_row(2 * g + 1, 1)
            return carry

        lax.fori_loop(0, BPW // 2, g_body, jnp.int32(0))

        pltpu.sync_copy(stage, out_hbm.at[pl.ds(base * LANES, BPW * LANES)])

    return pl.kernel(
        body,
        out_type=jax.ShapeDtypeStruct((B * LANES,), jnp.float32),
        mesh=mesh,
        compiler_params=pltpu.CompilerParams(
            needs_layout_passes=False, use_tc_tiling_on_sc=False),
        scratch_types=[
            pltpu.VMEM((BPW * LP2,), jnp.int32),
            pltpu.VMEM((LP2, DP), jnp.float32),
            pltpu.VMEM((LP2, DP), jnp.float32),
            pltpu.VMEM((2 * NCHUNK * LANES,), jnp.float32),
            pltpu.VMEM((INV_PAD,), jnp.float32),
            pltpu.VMEM((BPW * LANES,), jnp.float32),
            pltpu.SemaphoreType.DMA,
            pltpu.SemaphoreType.DMA,
        ],
    )


def kernel(candidates, head_mentions, emb_table, fc_w, fc_b):
    B, L = candidates.shape
    V, D = emb_table.shape
    DP = ((D + 7) // 8) * 8
    NCHUNK = DP // LANES

    table_p = jnp.pad(emb_table, ((0, 0), (0, DP - D)))

    w = fc_w[:, 0]
    halves = []
    for wa in (w[:D], w[D:]):
        wp = jnp.pad(wa, (0, DP - D))
        halves.append(wp.reshape(NCHUNK, LANES))
    wmat = jnp.stack(halves).reshape(-1)

    INV_PAD = ((L + 1 + 63) // 64) * 64
    inv_tab = jnp.where(
        jnp.arange(INV_PAD) <= L,
        1.0 / (jnp.float32(L) * jnp.arange(INV_PAD, dtype=jnp.float32)),
        0.0).astype(jnp.float32)

    LP2 = ((2 * L + 7) // 8) * 8
    toks = jnp.concatenate((candidates, head_mentions), axis=1)
    toks = jnp.pad(toks, ((0, 0), (0, LP2 - 2 * L))).reshape(-1)

    scores = _sc_scorer(B, L, V, DP, LP2)(
        toks, table_p, wmat, inv_tab)
    return scores.reshape(B, LANES)[:, :1] + fc_b


# speedup vs baseline: 12.4391x; 12.4391x over previous
import jax
import jax.numpy as jnp
from jax import lax
from jax.experimental import pallas as pl
from jax.experimental.pallas import tpu as pltpu
from jax.experimental.pallas import tpu_sc as plsc

PAD_ID = 0
LANES = 16
NUM_CORES = 2
NUM_SUBCORES = 16
NUM_WORKERS = NUM_CORES * NUM_SUBCORES
BN = 4096


def _project(emb_table, w2):
    V, D = emb_table.shape
    tt = emb_table.T
    nb = pl.cdiv(V, BN)

    def body(w_ref, t_ref, o_ref):
        o_ref[...] = jnp.dot(w_ref[...], t_ref[...],
                             preferred_element_type=jnp.float32)

    return pl.pallas_call(
        body,
        grid=(nb,),
        in_specs=[pl.BlockSpec((2, D), lambda i: (0, 0)),
                  pl.BlockSpec((D, BN), lambda i: (0, i))],
        out_specs=pl.BlockSpec((2, BN), lambda i: (0, i)),
        out_shape=jax.ShapeDtypeStruct((2, V), jnp.float32),
    )(w2, tt)


def _sc_scorer(B, L, V, LP2):
    BPW = B // NUM_WORKERS
    NCH = LP2 // LANES
    INV_PAD = ((L + 1 + 63) // 64) * 64

    mesh = plsc.VectorSubcoreMesh(
        core_axis_name="c", subcore_axis_name="s")

    def body(tok_hbm, p_hbm, inv_hbm, out_hbm,
             idx_v, vals0, vals1, inv_v, stage, sem0, sem1):
        wid = lax.axis_index("s") * NUM_CORES + lax.axis_index("c")
        base = wid * BPW
        pltpu.sync_copy(inv_hbm, inv_v)
        pltpu.sync_copy(tok_hbm.at[pl.ds(base * LP2, BPW * LP2)], idx_v)

        lane = jnp.arange(LANES, dtype=jnp.int32)
        zeros_f = jnp.zeros((LANES,), jnp.float32)
        zeros_i = jnp.zeros((LANES,), jnp.int32)
        cand_m = [(jnp.arange(k * LANES, (k + 1) * LANES) < L)
                  for k in range(NCH)]
        head_m = [((jnp.arange(k * LANES, (k + 1) * LANES) >= L)
                   & (jnp.arange(k * LANES, (k + 1) * LANES) < 2 * L))
                  for k in range(NCH)]
        sems = (sem0, sem1)
        bufs = (vals0, vals1)

        def gather(b, slot):
            pltpu.async_copy(p_hbm.at[idx_v.at[pl.ds(b * LP2, LP2)]],
                             bufs[slot], sems[slot])

        def wait(b, slot):
            pltpu.make_async_copy(p_hbm.at[idx_v.at[pl.ds(b * LP2, LP2)]],
                                  bufs[slot], sems[slot]).wait()

        def do_row(b, slot):
            wait(b, slot)
            vc = zeros_f
            vh = zeros_f
            nc = zeros_i
            nh = zeros_i
            for k in range(NCH):
                tok = plsc.load_gather(idx_v, [lane + (b * LP2 + k * LANES)])
                val = plsc.load_gather(bufs[slot], [lane + k * LANES])
                cm = jnp.asarray(cand_m[k]) & (tok != PAD_ID)
                hm = jnp.asarray(head_m[k]) & (tok != V)
                vc = vc + jnp.where(cm, val, 0.0)
                vh = vh + jnp.where(hm, val, 0.0)
                nc = nc + jnp.where(cm, 1, 0).astype(jnp.int32)
                nh = nh + jnp.where(hm, 1, 0).astype(jnp.int32)

            @pl.when(b + 2 < BPW)
            def _():
                gather(b + 2, slot)

            inv_c = plsc.load_gather(inv_v, [zeros_i + jnp.sum(nc)])
            inv_h = plsc.load_gather(inv_v, [zeros_i + jnp.sum(nh)])
            score = ((zeros_f + jnp.sum(vc)) * inv_c
                     + (zeros_f + jnp.sum(vh)) * inv_h)
            plsc.store_scatter(stage, [b * LANES + lane], score)

        gather(0, 0)
        gather(1, 1)

        def g_body(g, carry):
            do_row(2 * g, 0)
            do_row(2 * g + 1, 1)
            return carry

        lax.fori_loop(0, BPW // 2, g_body, jnp.int32(0))

        pltpu.sync_copy(stage, out_hbm.at[pl.ds(base * LANES, BPW * LANES)])

    return pl.kernel(
        body,
        out_type=jax.ShapeDtypeStruct((B * LANES,), jnp.float32),
        mesh=mesh,
        compiler_params=pltpu.CompilerParams(
            needs_layout_passes=False, use_tc_tiling_on_sc=False),
        scratch_types=[
            pltpu.VMEM((BPW * LP2,), jnp.int32),
            pltpu.VMEM((LP2,), jnp.float32),
            pltpu.VMEM((LP2,), jnp.float32),
            pltpu.VMEM((INV_PAD,), jnp.float32),
            pltpu.VMEM((BPW * LANES,), jnp.float32),
            pltpu.SemaphoreType.DMA,
            pltpu.SemaphoreType.DMA,
        ],
    )


def kernel(candidates, head_mentions, emb_table, fc_w, fc_b):
    B, L = candidates.shape
    V, D = emb_table.shape

    w2 = jnp.stack((fc_w[:D, 0], fc_w[D:, 0]))
    p = _project(emb_table, w2).reshape(-1)

    INV_PAD = ((L + 1 + 63) // 64) * 64
    inv_tab = jnp.where(
        jnp.arange(INV_PAD) <= L,
        1.0 / (jnp.float32(L) * jnp.arange(INV_PAD, dtype=jnp.float32)),
        0.0).astype(jnp.float32)

    LP2 = ((2 * L + 15) // 16) * 16
    toks = jnp.concatenate((candidates, head_mentions + V), axis=1)
    toks = jnp.pad(toks, ((0, 0), (0, LP2 - 2 * L))).reshape(-1)

    scores = _sc_scorer(B, L, V, LP2)(toks, p, inv_tab)
    return scores.reshape(B, LANES)[:, :1] + fc_b
